# baseline (speedup 1.0000x reference)
import jax
import jax.numpy as jnp
from jax.experimental import pallas as pl
from jax.experimental.pallas import tpu as pltpu

_OUT_FEATURES = 4096
_SIZE = 26
_LEVELS = 100
_LPAD = 128
_K = _SIZE * _LPAD
_NPLANES = 2
_WORDS = _OUT_FEATURES // _NPLANES
_LOW = 0.0
_HIGH = 1.0


def _pack_planes(bits_i32):
    acc = bits_i32[:, :_WORDS]
    for k in range(1, _NPLANES):
        acc = acc | (bits_i32[:, k * _WORDS : (k + 1) * _WORDS] << (5 * k))
    return acc


def _encode_kernel(x_ref, pos_ref, val_ref, out_ref, w2_ref):
    @pl.when(pl.program_id(0) == 0)
    def _build_w2():
        vp = _pack_planes(val_ref[...].astype(jnp.int32))
        vp = jnp.concatenate(
            [vp, jnp.zeros((_LPAD - _LEVELS, _WORDS), jnp.int32)], axis=0
        )
        pp = _pack_planes(pos_ref[...].astype(jnp.int32))
        for s in range(_SIZE):
            w2_ref[s * _LPAD : (s + 1) * _LPAD, :] = (
                pp[s : s + 1, :] ^ vp
            ).astype(jnp.bfloat16)

    x = x_ref[...]
    idx = jnp.clip(
        jnp.round((x - _LOW) / (_HIGH - _LOW) * (_LEVELS - 1)), 0, _LEVELS - 1
    ).astype(jnp.int32)
    lanes = jax.lax.broadcasted_iota(jnp.int32, (x.shape[0], _LPAD), 1)
    m = jnp.concatenate(
        [(idx[:, s : s + 1] == lanes) for s in range(_SIZE)], axis=1
    ).astype(jnp.bfloat16)

    counts = jnp.dot(m, w2_ref[...], preferred_element_type=jnp.float32).astype(jnp.int32)
    for k in range(_NPLANES):
        c = jax.lax.shift_right_logical(counts, 5 * k) & 31
        out_ref[:, k * _WORDS : (k + 1) * _WORDS] = (c >= 13).astype(jnp.uint8)


_BTILE = 512


def kernel(x, position_weight, value_weight):
    batch = x.shape[0]
    n_b = batch // _BTILE
    return pl.pallas_call(
        _encode_kernel,
        grid=(n_b,),
        in_specs=[
            pl.BlockSpec((_BTILE, _SIZE), lambda i: (i, 0)),
            pl.BlockSpec((_SIZE, _OUT_FEATURES), lambda i: (0, 0)),
            pl.BlockSpec((_LEVELS, _OUT_FEATURES), lambda i: (0, 0)),
        ],
        out_specs=pl.BlockSpec((_BTILE, _OUT_FEATURES), lambda i: (i, 0)),
        out_shape=jax.ShapeDtypeStruct((batch, _OUT_FEATURES), jnp.uint8),
        scratch_shapes=[pltpu.VMEM((_K, _WORDS), jnp.bfloat16)],
    )(x, position_weight, value_weight)


# speedup vs baseline: 1.1035x; 1.1035x over previous
import jax
import jax.numpy as jnp
from jax.experimental import pallas as pl

_OUT_FEATURES = 4096
_SIZE = 26
_LEVELS = 100
_LPAD = 128
_K = _SIZE * _LPAD
_NPLANES = 2
_WORDS = _OUT_FEATURES // _NPLANES
_LOW = 0.0
_HIGH = 1.0


def _pack_planes(bits_i32):
    acc = bits_i32[:, :_WORDS]
    for k in range(1, _NPLANES):
        acc = acc | (bits_i32[:, k * _WORDS : (k + 1) * _WORDS] << (5 * k))
    return acc


def _encode_kernel(x_ref, pos_ref, val_ref, out_ref):
    vp = _pack_planes(val_ref[...].astype(jnp.int32))
    vp = jnp.concatenate(
        [vp, jnp.zeros((_LPAD - _LEVELS, _WORDS), jnp.int32)], axis=0
    )
    pp = _pack_planes(pos_ref[...].astype(jnp.int32))
    w2 = jnp.concatenate(
        [(pp[s : s + 1, :] ^ vp).astype(jnp.bfloat16) for s in range(_SIZE)], axis=0
    )

    x = x_ref[...]
    idx = jnp.clip(
        jnp.round((x - _LOW) / (_HIGH - _LOW) * (_LEVELS - 1)), 0, _LEVELS - 1
    ).astype(jnp.int32)
    lanes = jax.lax.broadcasted_iota(jnp.int32, (x.shape[0], _LPAD), 1)
    m = jnp.concatenate(
        [(idx[:, s : s + 1] == lanes) for s in range(_SIZE)], axis=1
    ).astype(jnp.bfloat16)

    counts = jnp.dot(m, w2, preferred_element_type=jnp.float32).astype(jnp.int32)
    out_ref[:, :_WORDS] = ((counts & 31) >= 13).astype(jnp.uint8)
    out_ref[:, _WORDS:] = (counts >= 13 * 32).astype(jnp.uint8)


def kernel(x, position_weight, value_weight):
    batch = x.shape[0]
    return pl.pallas_call(
        _encode_kernel,
        out_shape=jax.ShapeDtypeStruct((batch, _OUT_FEATURES), jnp.uint8),
    )(x, position_weight, value_weight)
